# baseline (speedup 1.0000x reference)
import functools

import jax
import jax.numpy as jnp
from jax import lax
from jax.experimental import pallas as pl
from jax.experimental.pallas import tpu as pltpu
from jax.experimental.pallas import tpu_sc as plsc

NUM_CORES = 2
NUM_SUBCORES = 16
LANES = 16
NW = NUM_CORES * NUM_SUBCORES

BATCH = 16384
ROWS = BATCH // NW
GROUPS = ROWS // LANES
NT = 100
NI, NV, NTG = 6, 10, 15
IDX_PAD = 128

_mesh = plsc.VectorSubcoreMesh(core_axis_name="c", subcore_axis_name="s")


def _lane_gather(x, idx):
    dnums = lax.GatherDimensionNumbers(
        offset_dims=(), collapsed_slice_dims=(0,), start_index_map=(0,))
    return lax.gather(x, idx[:, None], dnums, slice_sizes=(1,),
                      mode=lax.GatherScatterMode.PROMISE_IN_BOUNDS)


@functools.partial(
    pl.kernel,
    out_type=jax.ShapeDtypeStruct((BATCH * NT,), jnp.float32),
    mesh=_mesh,
    compiler_params=pltpu.CompilerParams(needs_layout_passes=False),
    scratch_types=[
        pltpu.VMEM((ROWS * NI,), jnp.float32),
        pltpu.VMEM((ROWS * NV,), jnp.float32),
        pltpu.VMEM((ROWS * NTG,), jnp.float32),
        pltpu.VMEM((ROWS * NT,), jnp.float32),
        pltpu.VMEM((IDX_PAD,), jnp.int32),
        pltpu.VMEM((IDX_PAD,), jnp.int32),
        pltpu.VMEM((IDX_PAD,), jnp.int32),
        pltpu.VMEM((IDX_PAD,), jnp.float32),
        pltpu.VMEM((LANES,), jnp.float32),
    ],
)
def _sc_combine(inst_hbm, verb_hbm, targ_hbm, icol_hbm, vcol_hbm, tcol_hbm,
                bias_hbm, w_hbm, out_hbm,
                inst_v, verb_v, targ_v, out_v, icol_v, vcol_v, tcol_v,
                bias_v, w_v):
    wid = lax.axis_index("s") * NUM_CORES + lax.axis_index("c")

    pltpu.sync_copy(inst_hbm.at[pl.ds(wid * (ROWS * NI), ROWS * NI)], inst_v)
    pltpu.sync_copy(verb_hbm.at[pl.ds(wid * (ROWS * NV), ROWS * NV)], verb_v)
    pltpu.sync_copy(targ_hbm.at[pl.ds(wid * (ROWS * NTG), ROWS * NTG)], targ_v)
    pltpu.sync_copy(icol_hbm, icol_v)
    pltpu.sync_copy(vcol_hbm, vcol_v)
    pltpu.sync_copy(tcol_hbm, tcol_v)
    pltpu.sync_copy(bias_hbm, bias_v)
    pltpu.sync_copy(w_hbm, w_v)

    zero = jnp.zeros((LANES,), jnp.int32)
    wvec = w_v[...]
    w0 = _lane_gather(wvec, zero)
    w1 = _lane_gather(wvec, zero + 1)
    w2 = _lane_gather(wvec, zero + 2)
    iota = lax.iota(jnp.int32, LANES)

    def t_body(t, carry):
        ti = jnp.full((LANES,), t, jnp.int32)
        ci = plsc.load_gather(icol_v, [ti])
        cv = plsc.load_gather(vcol_v, [ti])
        ct = plsc.load_gather(tcol_v, [ti])
        bt = plsc.load_gather(bias_v, [ti])
        idx0 = (iota * NI + ci, iota * NV + cv, iota * NTG + ct,
                iota * NT + ti)

        def g_body(g, idx):
            idx_i, idx_v, idx_t, idx_o = idx
            li = plsc.load_gather(inst_v, [idx_i])
            lv = plsc.load_gather(verb_v, [idx_v])
            lt = plsc.load_gather(targ_v, [idx_t])
            acc = bt + w0 * li + w1 * lv + w2 * lt
            plsc.store_scatter(out_v, [idx_o], acc)
            return (idx_i + LANES * NI, idx_v + LANES * NV,
                    idx_t + LANES * NTG, idx_o + LANES * NT)

        lax.fori_loop(0, GROUPS, g_body, idx0)
        return carry

    lax.fori_loop(0, NT, t_body, 0)

    pltpu.sync_copy(out_v, out_hbm.at[pl.ds(wid * (ROWS * NT), ROWS * NT)])


def kernel(instrument_logits, verb_logits, target_logits, triplet_to_ivt,
           triplet_bias, component_weights):
    ivt = triplet_to_ivt.astype(jnp.int32)
    icol = jnp.zeros((IDX_PAD,), jnp.int32).at[:NT].set(ivt[:, 0])
    vcol = jnp.zeros((IDX_PAD,), jnp.int32).at[:NT].set(ivt[:, 1])
    tcol = jnp.zeros((IDX_PAD,), jnp.int32).at[:NT].set(ivt[:, 2])
    bias = jnp.zeros((IDX_PAD,), jnp.float32).at[:NT].set(
        triplet_bias.astype(jnp.float32))
    w = jax.nn.softmax(component_weights.astype(jnp.float32))
    w_pad = jnp.zeros((LANES,), jnp.float32).at[:3].set(w)
    out = _sc_combine(instrument_logits.reshape(-1), verb_logits.reshape(-1),
                      target_logits.reshape(-1),
                      icol, vcol, tcol, bias, w_pad)
    return out.reshape(BATCH, NT)


# speedup vs baseline: 1.2001x; 1.2001x over previous
import functools

import jax
import jax.numpy as jnp
from jax import lax
from jax.experimental import pallas as pl
from jax.experimental.pallas import tpu as pltpu
from jax.experimental.pallas import tpu_sc as plsc

NUM_CORES = 2
NUM_SUBCORES = 16
LANES = 16
NW = NUM_CORES * NUM_SUBCORES

BATCH = 16384
ROWS = BATCH // NW
GROUPS = ROWS // LANES
NT = 100
NI, NV, NTG = 6, 10, 15
IDX_PAD = 128

_mesh = plsc.VectorSubcoreMesh(core_axis_name="c", subcore_axis_name="s")


def _lane_gather(x, idx):
    dnums = lax.GatherDimensionNumbers(
        offset_dims=(), collapsed_slice_dims=(0,), start_index_map=(0,))
    return lax.gather(x, idx[:, None], dnums, slice_sizes=(1,),
                      mode=lax.GatherScatterMode.PROMISE_IN_BOUNDS)


@functools.partial(
    pl.kernel,
    out_type=jax.ShapeDtypeStruct((BATCH * NT,), jnp.float32),
    mesh=_mesh,
    compiler_params=pltpu.CompilerParams(needs_layout_passes=False),
    scratch_types=[
        pltpu.VMEM((ROWS * NI,), jnp.float32),
        pltpu.VMEM((ROWS * NV,), jnp.float32),
        pltpu.VMEM((ROWS * NTG,), jnp.float32),
        pltpu.VMEM((ROWS * NT,), jnp.float32),
        pltpu.VMEM((IDX_PAD,), jnp.int32),
        pltpu.VMEM((IDX_PAD,), jnp.int32),
        pltpu.VMEM((IDX_PAD,), jnp.int32),
        pltpu.VMEM((IDX_PAD,), jnp.float32),
        pltpu.VMEM((LANES,), jnp.float32),
    ],
)
def _sc_combine(inst_hbm, verb_hbm, targ_hbm, icol_hbm, vcol_hbm, tcol_hbm,
                bias_hbm, w_hbm, out_hbm,
                inst_v, verb_v, targ_v, out_v, icol_v, vcol_v, tcol_v,
                bias_v, w_v):
    wid = lax.axis_index("s") * NUM_CORES + lax.axis_index("c")

    pltpu.sync_copy(inst_hbm.at[pl.ds(wid * (ROWS * NI), ROWS * NI)], inst_v)
    pltpu.sync_copy(verb_hbm.at[pl.ds(wid * (ROWS * NV), ROWS * NV)], verb_v)
    pltpu.sync_copy(targ_hbm.at[pl.ds(wid * (ROWS * NTG), ROWS * NTG)], targ_v)
    pltpu.sync_copy(icol_hbm, icol_v)
    pltpu.sync_copy(vcol_hbm, vcol_v)
    pltpu.sync_copy(tcol_hbm, tcol_v)
    pltpu.sync_copy(bias_hbm, bias_v)
    pltpu.sync_copy(w_hbm, w_v)

    zero = jnp.zeros((LANES,), jnp.int32)
    wvec = w_v[...]
    w0 = _lane_gather(wvec, zero)
    w1 = _lane_gather(wvec, zero + 1)
    w2 = _lane_gather(wvec, zero + 2)
    iota = lax.iota(jnp.int32, LANES)

    def t_body(t, carry):
        ti = jnp.full((LANES,), t, jnp.int32)
        ci = plsc.load_gather(icol_v, [ti])
        cv = plsc.load_gather(vcol_v, [ti])
        ct = plsc.load_gather(tcol_v, [ti])
        bt = plsc.load_gather(bias_v, [ti])
        base_i = iota * NI + ci
        base_v = iota * NV + cv
        base_t = iota * NTG + ct
        base_o = iota * NT + ti

        @plsc.parallel_loop(0, GROUPS, unroll=8)
        def g_body(g):
            li = plsc.load_gather(inst_v, [base_i + g * (LANES * NI)])
            lv = plsc.load_gather(verb_v, [base_v + g * (LANES * NV)])
            lt = plsc.load_gather(targ_v, [base_t + g * (LANES * NTG)])
            acc = bt + w0 * li + w1 * lv + w2 * lt
            plsc.store_scatter(out_v, [base_o + g * (LANES * NT)], acc)

        return carry

    lax.fori_loop(0, NT, t_body, 0)

    pltpu.sync_copy(out_v, out_hbm.at[pl.ds(wid * (ROWS * NT), ROWS * NT)])


def kernel(instrument_logits, verb_logits, target_logits, triplet_to_ivt,
           triplet_bias, component_weights):
    ivt = triplet_to_ivt.astype(jnp.int32)
    icol = jnp.zeros((IDX_PAD,), jnp.int32).at[:NT].set(ivt[:, 0])
    vcol = jnp.zeros((IDX_PAD,), jnp.int32).at[:NT].set(ivt[:, 1])
    tcol = jnp.zeros((IDX_PAD,), jnp.int32).at[:NT].set(ivt[:, 2])
    bias = jnp.zeros((IDX_PAD,), jnp.float32).at[:NT].set(
        triplet_bias.astype(jnp.float32))
    w = jax.nn.softmax(component_weights.astype(jnp.float32))
    w_pad = jnp.zeros((LANES,), jnp.float32).at[:3].set(w)
    out = _sc_combine(instrument_logits.reshape(-1), verb_logits.reshape(-1),
                      target_logits.reshape(-1),
                      icol, vcol, tcol, bias, w_pad)
    return out.reshape(BATCH, NT)
